# baseline (speedup 1.0000x reference)
import functools

import jax
import jax.numpy as jnp
from jax import lax
from jax.experimental import pallas as pl
from jax.experimental.pallas import tpu as pltpu
from jax.experimental.pallas import tpu_sc as plsc

_NUM_BINS = 1000000
_HASH_MULT = 2654435761
_EMBED_DIM = 32
_NB = 4
_NC = 2
_NS = 16
_LANES = 16
_WB = 4
_PADBLK = 1600


def _copy_body(x_ref, o_ref):
    o_ref[...] = x_ref[...]


def _stage_table(table):
    v, d = table.shape
    return pl.pallas_call(
        _copy_body,
        grid=(v // _PADBLK,),
        in_specs=[pl.BlockSpec((_PADBLK, d), lambda i: (i, 0))],
        out_specs=pl.BlockSpec((_PADBLK, d), lambda i: (i, 0)),
        out_shape=jax.ShapeDtypeStruct((v, d), table.dtype),
    )(table)


def _sc_hash_gather(t128, raw_flat, b, f):
    mesh = plsc.VectorSubcoreMesh(core_axis_name="core", subcore_axis_name="subcore")
    out_type = jax.ShapeDtypeStruct((b, 32, 128), t128.dtype)
    n_tiles = _NC * _NS
    b_per_tile = b // n_tiles
    n_win = b_per_tile // _WB
    n_outer = n_win // _NB
    wrows = _WB * f

    @functools.partial(
        pl.kernel,
        out_type=out_type,
        mesh=mesh,
        scratch_types=(
            [
                pltpu.VMEM((_NB, 128), jnp.int32),
                pltpu.VMEM((_NB, 128), jnp.int32),
                pltpu.VMEM((_NB, wrows, _EMBED_DIM), jnp.float32),
            ]
            + [pltpu.SemaphoreType.DMA] * (3 * _NB)
        ),
        compiler_params=pltpu.CompilerParams(use_tc_tiling_on_sc=False),
    )
    def k(t_hbm, in_hbm, out_hbm, raw_v, idx_v, rows_v, *sems):
        sem_raw = sems[0:_NB]
        sem_g = sems[_NB : 2 * _NB]
        sem_out = sems[2 * _NB : 3 * _NB]
        wid = lax.axis_index("subcore") * _NC + lax.axis_index("core")
        row0 = wid * b_per_tile * f
        bt0 = wid * b_per_tile

        def start_raw(w, u):
            pltpu.async_copy(
                in_hbm.at[pl.ds(row0 + w * wrows, wrows)],
                raw_v.at[u, pl.ds(0, wrows)],
                sem_raw[u],
            )

        def wait_raw(u):
            pltpu.make_async_copy(
                in_hbm.at[pl.ds(0, wrows)],
                raw_v.at[u, pl.ds(0, wrows)],
                sem_raw[u],
            ).wait()

        def start_gather(u):
            pltpu.async_copy(
                t_hbm.at[idx_v.at[u, pl.ds(0, wrows)]], rows_v.at[u], sem_g[u]
            )

        def wait_gather(u):
            pltpu.make_async_copy(
                t_hbm.at[idx_v.at[u, pl.ds(0, wrows)]], rows_v.at[u], sem_g[u]
            ).wait()

        def start_out(w, u):
            for bq in range(_WB):
                pltpu.async_copy(
                    rows_v.at[u, pl.ds(bq * f, f), :],
                    out_hbm.at[bt0 + w * _WB + bq, pl.ds(0, f), pl.ds(0, _EMBED_DIM)],
                    sem_out[u],
                )

        def wait_out(u):
            for bq in range(_WB):
                pltpu.make_async_copy(
                    rows_v.at[u, pl.ds(bq * f, f), :],
                    out_hbm.at[bq, pl.ds(0, f), pl.ds(0, _EMBED_DIM)],
                    sem_out[u],
                ).wait()

        def hash_window(u):
            for c in range(128 // _LANES):
                sl = pl.ds(c * _LANES, _LANES)
                v = raw_v[u, sl].astype(jnp.uint32)
                h = (v * jnp.uint32(_HASH_MULT)) % jnp.uint32(_NUM_BINS)
                idx_v[u, sl] = h.astype(jnp.int32)

        for u in range(_NB):
            start_raw(u, u)

        @pl.loop(0, n_outer)
        def _(o):
            for u in range(_NB):
                w = o * _NB + u

                @pl.when(o > 0)
                def _():
                    wait_out(u)

                wait_raw(u)
                hash_window(u)
                start_gather(u)

                up = (u - 1) % _NB
                if u > 0:
                    wait_gather(up)
                    start_out(w - 1, up)
                else:

                    @pl.when(o > 0)
                    def _():
                        wait_gather(up)
                        start_out(w - 1, up)

                @pl.when(o < n_outer - 1)
                def _():
                    start_raw(w + _NB, u)

        lu = (n_win - 1) % _NB
        wait_gather(lu)
        start_out(n_win - 1, lu)
        for u in range(_NB):
            wait_out(u)

    return k(t128, raw_flat)


def kernel(inputs, table):
    b, f = inputs.shape
    n = b * f
    out3 = _sc_hash_gather(_stage_table(table), inputs.reshape(n), b, f)
    return out3[:, :f, :_EMBED_DIM]


# speedup vs baseline: 2.1055x; 2.1055x over previous
import functools

import jax
import jax.numpy as jnp
from jax import lax
from jax.experimental import pallas as pl
from jax.experimental.pallas import tpu as pltpu
from jax.experimental.pallas import tpu_sc as plsc

_NUM_BINS = 1000000
_HASH_MULT = 2654435761
_EMBED_DIM = 32
_NB = 4
_NC = 2
_NS = 16
_LANES = 16
_WB = 4
_PADBLK = 1600


def _pad_body(x_ref, o_ref):
    o_ref[:, : _EMBED_DIM] = x_ref[...]


def _widen_table(table):
    v, d = table.shape
    return pl.pallas_call(
        _pad_body,
        grid=(v // _PADBLK,),
        in_specs=[pl.BlockSpec((_PADBLK, d), lambda i: (i, 0))],
        out_specs=pl.BlockSpec((_PADBLK, 128), lambda i: (i, 0)),
        out_shape=jax.ShapeDtypeStruct((v, 128), table.dtype),
    )(table)


def _sc_hash_gather(t128, raw_flat, b, f):
    mesh = plsc.VectorSubcoreMesh(core_axis_name="core", subcore_axis_name="subcore")
    out_type = jax.ShapeDtypeStruct((b, 32, 128), t128.dtype)
    n_tiles = _NC * _NS
    b_per_tile = b // n_tiles
    n_win = b_per_tile // _WB
    n_outer = n_win // _NB
    wrows = _WB * f

    @functools.partial(
        pl.kernel,
        out_type=out_type,
        mesh=mesh,
        scratch_types=(
            [
                pltpu.VMEM((_NB, 128), jnp.int32),
                pltpu.VMEM((_NB, 128), jnp.int32),
                pltpu.VMEM((_NB, wrows, _EMBED_DIM), jnp.float32),
            ]
            + [pltpu.SemaphoreType.DMA] * (3 * _NB)
        ),
        compiler_params=pltpu.CompilerParams(use_tc_tiling_on_sc=False),
    )
    def k(t_hbm, in_hbm, out_hbm, raw_v, idx_v, rows_v, *sems):
        sem_raw = sems[0:_NB]
        sem_g = sems[_NB : 2 * _NB]
        sem_out = sems[2 * _NB : 3 * _NB]
        wid = lax.axis_index("subcore") * _NC + lax.axis_index("core")
        row0 = wid * b_per_tile * f
        bt0 = wid * b_per_tile

        def start_raw(w, u):
            pltpu.async_copy(
                in_hbm.at[pl.ds(row0 + w * wrows, wrows)],
                raw_v.at[u, pl.ds(0, wrows)],
                sem_raw[u],
            )

        def wait_raw(u):
            pltpu.make_async_copy(
                in_hbm.at[pl.ds(0, wrows)],
                raw_v.at[u, pl.ds(0, wrows)],
                sem_raw[u],
            ).wait()

        def start_gather(u):
            pltpu.async_copy(
                t_hbm.at[idx_v.at[u, pl.ds(0, wrows)]], rows_v.at[u], sem_g[u]
            )

        def wait_gather(u):
            pltpu.make_async_copy(
                t_hbm.at[idx_v.at[u, pl.ds(0, wrows)]], rows_v.at[u], sem_g[u]
            ).wait()

        def start_out(w, u):
            for bq in range(_WB):
                pltpu.async_copy(
                    rows_v.at[u, pl.ds(bq * f, f), :],
                    out_hbm.at[bt0 + w * _WB + bq, pl.ds(0, f), pl.ds(0, _EMBED_DIM)],
                    sem_out[u],
                )

        def wait_out(u):
            for bq in range(_WB):
                pltpu.make_async_copy(
                    rows_v.at[u, pl.ds(bq * f, f), :],
                    out_hbm.at[bq, pl.ds(0, f), pl.ds(0, _EMBED_DIM)],
                    sem_out[u],
                ).wait()

        def hash_window(u):
            for c in range(128 // _LANES):
                sl = pl.ds(c * _LANES, _LANES)
                v = raw_v[u, sl].astype(jnp.uint32)
                h = (v * jnp.uint32(_HASH_MULT)) % jnp.uint32(_NUM_BINS)
                idx_v[u, sl] = h.astype(jnp.int32)

        for u in range(_NB):
            start_raw(u, u)

        @pl.loop(0, n_outer)
        def _(o):
            for u in range(_NB):
                w = o * _NB + u

                @pl.when(o > 0)
                def _():
                    wait_out(u)

                wait_raw(u)
                hash_window(u)
                start_gather(u)

                up = (u - 2) % _NB
                if u >= 2:
                    wait_gather(up)
                    start_out(w - 2, up)
                else:

                    @pl.when(o > 0)
                    def _():
                        wait_gather(up)
                        start_out(w - 2, up)

                @pl.when(o < n_outer - 1)
                def _():
                    start_raw(w + _NB, u)

        for w in (n_win - 2, n_win - 1):
            lu = w % _NB
            wait_gather(lu)
            start_out(w, lu)
        for u in range(_NB):
            wait_out(u)

    return k(t128, raw_flat)


def kernel(inputs, table):
    b, f = inputs.shape
    n = b * f
    out3 = _sc_hash_gather(table, inputs.reshape(n), b, f)
    return out3[:, :f, :_EMBED_DIM]


# speedup vs baseline: 2.1202x; 1.0070x over previous
import functools

import jax
import jax.numpy as jnp
from jax import lax
from jax.experimental import pallas as pl
from jax.experimental.pallas import tpu as pltpu
from jax.experimental.pallas import tpu_sc as plsc

_NUM_BINS = 1000000
_HASH_MULT = 2654435761
_EMBED_DIM = 32
_NB = 4
_NC = 2
_NS = 16
_LANES = 16
_WB = 4
_PADBLK = 1600


def _pad_body(x_ref, o_ref):
    o_ref[:, : _EMBED_DIM] = x_ref[...]


def _widen_table(table):
    v, d = table.shape
    return pl.pallas_call(
        _pad_body,
        grid=(v // _PADBLK,),
        in_specs=[pl.BlockSpec((_PADBLK, d), lambda i: (i, 0))],
        out_specs=pl.BlockSpec((_PADBLK, 128), lambda i: (i, 0)),
        out_shape=jax.ShapeDtypeStruct((v, 128), table.dtype),
    )(table)


def _sc_hash_gather(t128, raw_flat, b, f):
    mesh = plsc.VectorSubcoreMesh(core_axis_name="core", subcore_axis_name="subcore")
    out_type = jax.ShapeDtypeStruct((b, 32, 128), t128.dtype)
    n_tiles = _NC * _NS
    b_per_tile = b // n_tiles
    n_win = b_per_tile // _WB
    n_outer = n_win // _NB
    wrows = _WB * f

    @functools.partial(
        pl.kernel,
        out_type=out_type,
        mesh=mesh,
        scratch_types=(
            [
                pltpu.VMEM((_NB, 128), jnp.int32),
                pltpu.VMEM((_NB, 128), jnp.int32),
                pltpu.VMEM((_NB, wrows, _EMBED_DIM), jnp.float32),
            ]
            + [pltpu.SemaphoreType.DMA] * (3 * _NB)
        ),
        compiler_params=pltpu.CompilerParams(use_tc_tiling_on_sc=False),
    )
    def k(t_hbm, in_hbm, out_hbm, raw_v, idx_v, rows_v, *sems):
        sem_raw = sems[0:_NB]
        sem_g = sems[_NB : 2 * _NB]
        sem_out = sems[2 * _NB : 3 * _NB]
        wid = lax.axis_index("subcore") * _NC + lax.axis_index("core")
        row0 = wid * b_per_tile * f
        bt0 = wid * b_per_tile

        def start_raw(w, u):
            pltpu.async_copy(
                in_hbm.at[pl.ds(row0 + w * wrows, wrows)],
                raw_v.at[u, pl.ds(0, wrows)],
                sem_raw[u],
            )

        def wait_raw(u):
            pltpu.make_async_copy(
                in_hbm.at[pl.ds(0, wrows)],
                raw_v.at[u, pl.ds(0, wrows)],
                sem_raw[u],
            ).wait()

        def start_gather(u):
            pltpu.async_copy(
                t_hbm.at[idx_v.at[u, pl.ds(0, wrows)]], rows_v.at[u], sem_g[u]
            )

        def wait_gather(u):
            pltpu.make_async_copy(
                t_hbm.at[idx_v.at[u, pl.ds(0, wrows)]], rows_v.at[u], sem_g[u]
            ).wait()

        def start_out(w, u):
            for bq in range(_WB):
                pltpu.async_copy(
                    rows_v.at[u, pl.ds(bq * f, f), :],
                    out_hbm.at[bt0 + w * _WB + bq, pl.ds(0, f), pl.ds(0, _EMBED_DIM)],
                    sem_out[u],
                )

        def wait_out(u):
            for bq in range(_WB):
                pltpu.make_async_copy(
                    rows_v.at[u, pl.ds(bq * f, f), :],
                    out_hbm.at[bq, pl.ds(0, f), pl.ds(0, _EMBED_DIM)],
                    sem_out[u],
                ).wait()

        def hash_window(u):
            for c in range(128 // _LANES):
                sl = pl.ds(c * _LANES, _LANES)
                v = raw_v[u, sl].astype(jnp.uint32)
                h = (v * jnp.uint32(_HASH_MULT)) % jnp.uint32(_NUM_BINS)
                idx_v[u, sl] = h.astype(jnp.int32)

        for u in range(_NB):
            start_raw(u, u)

        @pl.loop(0, n_outer)
        def _(o):
            for u in range(_NB):
                w = o * _NB + u

                @pl.when(o > 0)
                def _():
                    wait_out(u)

                wait_raw(u)
                hash_window(u)
                start_gather(u)

                up = (u - 3) % _NB
                if u >= 3:
                    wait_gather(up)
                    start_out(w - 3, up)
                else:

                    @pl.when(o > 0)
                    def _():
                        wait_gather(up)
                        start_out(w - 3, up)

                @pl.when(o < n_outer - 1)
                def _():
                    start_raw(w + _NB, u)

        for w in (n_win - 3, n_win - 2, n_win - 1):
            lu = w % _NB
            wait_gather(lu)
            start_out(w, lu)
        for u in range(_NB):
            wait_out(u)

    return k(t128, raw_flat)


def kernel(inputs, table):
    b, f = inputs.shape
    n = b * f
    out3 = _sc_hash_gather(table, inputs.reshape(n), b, f)
    return out3[:, :f, :_EMBED_DIM]


# speedup vs baseline: 2.1205x; 1.0001x over previous
import functools

import jax
import jax.numpy as jnp
from jax import lax
from jax.experimental import pallas as pl
from jax.experimental.pallas import tpu as pltpu
from jax.experimental.pallas import tpu_sc as plsc

_NUM_BINS = 1000000
_HASH_MULT = 2654435761
_EMBED_DIM = 32
_NB = 4
_NC = 2
_NS = 16
_LANES = 16
_WB = 4


def _sc_hash_gather(t128, raw_flat, b, f):
    mesh = plsc.VectorSubcoreMesh(core_axis_name="core", subcore_axis_name="subcore")
    out_type = jax.ShapeDtypeStruct((b, 32, 128), t128.dtype)
    n_tiles = _NC * _NS
    b_per_tile = b // n_tiles
    n_win = b_per_tile // _WB
    n_outer = n_win // _NB
    wrows = _WB * f

    @functools.partial(
        pl.kernel,
        out_type=out_type,
        mesh=mesh,
        scratch_types=(
            [
                pltpu.VMEM((_NB, 128), jnp.int32),
                pltpu.VMEM((_NB, 128), jnp.int32),
                pltpu.VMEM((_NB, wrows, _EMBED_DIM), jnp.float32),
            ]
            + [pltpu.SemaphoreType.DMA] * (3 * _NB)
        ),
        compiler_params=pltpu.CompilerParams(use_tc_tiling_on_sc=False),
    )
    def k(t_hbm, in_hbm, out_hbm, raw_v, idx_v, rows_v, *sems):
        sem_raw = sems[0:_NB]
        sem_g = sems[_NB : 2 * _NB]
        sem_out = sems[2 * _NB : 3 * _NB]
        wid = lax.axis_index("subcore") * _NC + lax.axis_index("core")
        row0 = wid * b_per_tile * f
        bt0 = wid * b_per_tile

        def start_raw(w, u):
            pltpu.async_copy(
                in_hbm.at[pl.ds(row0 + w * wrows, wrows)],
                raw_v.at[u, pl.ds(0, wrows)],
                sem_raw[u],
            )

        def wait_raw(u):
            pltpu.make_async_copy(
                in_hbm.at[pl.ds(0, wrows)],
                raw_v.at[u, pl.ds(0, wrows)],
                sem_raw[u],
            ).wait()

        def start_gather(u):
            pltpu.async_copy(
                t_hbm.at[idx_v.at[u, pl.ds(0, wrows)]], rows_v.at[u], sem_g[u]
            )

        def wait_gather(u):
            pltpu.make_async_copy(
                t_hbm.at[idx_v.at[u, pl.ds(0, wrows)]], rows_v.at[u], sem_g[u]
            ).wait()

        def start_out(w, u):
            for bq in range(_WB):
                pltpu.async_copy(
                    rows_v.at[u, pl.ds(bq * f, f), :],
                    out_hbm.at[bt0 + w * _WB + bq, pl.ds(0, f), pl.ds(0, _EMBED_DIM)],
                    sem_out[u],
                )

        def wait_out(u):
            for bq in range(_WB):
                pltpu.make_async_copy(
                    rows_v.at[u, pl.ds(bq * f, f), :],
                    out_hbm.at[bq, pl.ds(0, f), pl.ds(0, _EMBED_DIM)],
                    sem_out[u],
                ).wait()

        def hash_window(u):
            for c in range(128 // _LANES):
                sl = pl.ds(c * _LANES, _LANES)
                v = raw_v[u, sl].astype(jnp.uint32)
                h = (v * jnp.uint32(_HASH_MULT)) % jnp.uint32(_NUM_BINS)
                idx_v[u, sl] = h.astype(jnp.int32)

        for u in range(_NB):
            start_raw(u, u)

        @pl.loop(0, n_outer)
        def _(o):
            for u in range(_NB):
                w = o * _NB + u

                @pl.when(o > 0)
                def _():
                    wait_out(u)

                wait_raw(u)
                hash_window(u)
                start_gather(u)

                up = (u - 3) % _NB
                if u >= 3:
                    wait_gather(up)
                    start_out(w - 3, up)
                else:

                    @pl.when(o > 0)
                    def _():
                        wait_gather(up)
                        start_out(w - 3, up)

                @pl.when(o < n_outer - 1)
                def _():
                    start_raw(w + _NB, u)

        for w in (n_win - 3, n_win - 2, n_win - 1):
            lu = w % _NB
            wait_gather(lu)
            start_out(w, lu)
        for u in range(_NB):
            wait_out(u)

    return k(t128, raw_flat)


def kernel(inputs, table):
    b, f = inputs.shape
    n = b * f
    out3 = _sc_hash_gather(table, inputs.reshape(n), b, f)
    return out3[:, :f, :_EMBED_DIM]
